# baseline (speedup 1.0000x reference)
import functools

import jax
import jax.numpy as jnp
from jax import lax
from jax.experimental import pallas as pl

CONF_THRES = 0.25
NEG_INF = float("-inf")


def _body(pred_ref, det_ref, mask_ref, scores_ref):
    x = pred_ref[0]
    N, D = x.shape
    C = D - 5

    li = lax.broadcasted_iota(jnp.int32, (N, D), 1)

    conf = jnp.max(jnp.where(li == 4, x, NEG_INF), axis=1, keepdims=True)
    x1 = jnp.max(jnp.where(li == 0, x, NEG_INF), axis=1, keepdims=True)
    y1 = jnp.max(jnp.where(li == 1, x, NEG_INF), axis=1, keepdims=True)
    x2 = jnp.max(jnp.where(li == 2, x, NEG_INF), axis=1, keepdims=True)
    y2 = jnp.max(jnp.where(li == 3, x, NEG_INF), axis=1, keepdims=True)

    is_cls = li >= 5
    class_conf = jnp.max(jnp.where(is_cls, x, NEG_INF), axis=1,
                         keepdims=True)
    eq = (x == class_conf) & is_cls
    class_pred = jnp.min(jnp.where(eq, li - 5, jnp.int32(1 << 30)),
                         axis=1, keepdims=True).astype(jnp.float32)

    conf_mask = conf >= CONF_THRES
    above = jnp.sum(conf_mask.astype(jnp.int32))

    iota = lax.broadcasted_iota(jnp.int32, (N, 1), 0)
    rem = conf
    fb = jnp.zeros((N, 1), jnp.bool_)
    for _ in range(5):
        m = jnp.max(rem)
        j = jnp.min(jnp.where(rem == m, iota, jnp.int32(N)))
        hit = iota == j
        fb = fb | hit
        rem = jnp.where(hit, NEG_INF, rem)

    no_above = above == 0
    keep = (fb & no_above) | (conf_mask & jnp.logical_not(no_above))
    valid = (x2 > x1) & (y2 > y1)
    final = keep & valid

    mask_ref[0] = final
    scores_ref[0] = jnp.where(final, conf, NEG_INF)

    li7 = lax.broadcasted_iota(jnp.int32, (N, 7), 1)
    det = x[:, :7]
    det = jnp.where(li7 == 5, class_conf, det)
    det = jnp.where(li7 == 6, class_pred, det)
    det_ref[0] = det


@jax.jit
def kernel(prediction):
    B, N, D = prediction.shape
    det, mask, scores = pl.pallas_call(
        _body,
        grid=(B,),
        in_specs=[pl.BlockSpec((1, N, D), lambda b: (b, 0, 0))],
        out_specs=[
            pl.BlockSpec((1, N, 7), lambda b: (b, 0, 0)),
            pl.BlockSpec((1, N, 1), lambda b: (b, 0, 0)),
            pl.BlockSpec((1, N, 1), lambda b: (b, 0, 0)),
        ],
        out_shape=[
            jax.ShapeDtypeStruct((B, N, 7), jnp.float32),
            jax.ShapeDtypeStruct((B, N, 1), jnp.bool_),
            jax.ShapeDtypeStruct((B, N, 1), jnp.float32),
        ],
    )(prediction)
    return det, mask[:, :, 0], scores[:, :, 0]


# speedup vs baseline: 1.4019x; 1.4019x over previous
import functools

import jax
import jax.numpy as jnp
from jax import lax
from jax.experimental import pallas as pl

CONF_THRES = 0.25
NEG_INF = float("-inf")


def _body(pred_ref, det_ref, mask_ref, scores_ref):
    x = pred_ref[0]
    N, D = x.shape
    C = D - 5

    x1 = x[:, 0:1]
    y1 = x[:, 1:2]
    x2 = x[:, 2:3]
    y2 = x[:, 3:4]
    conf = x[:, 4:5]

    cls = x[:, 5:]
    class_conf = jnp.max(cls, axis=1, keepdims=True)
    ci = lax.broadcasted_iota(jnp.int32, (N, C), 1)
    class_pred = jnp.min(jnp.where(cls == class_conf, ci, jnp.int32(1 << 30)),
                         axis=1, keepdims=True).astype(jnp.float32)

    conf_mask = conf >= CONF_THRES
    above = jnp.sum(conf_mask.astype(jnp.int32))
    valid = (x2 > x1) & (y2 > y1)

    final = conf_mask & valid
    mask_ref[0] = final
    scores_ref[0] = jnp.where(final, conf, NEG_INF)

    li7 = lax.broadcasted_iota(jnp.int32, (N, 7), 1)
    det = x[:, :7]
    det = jnp.where(li7 == 5, class_conf, det)
    det = jnp.where(li7 == 6, class_pred, det)
    det_ref[0] = det

    @pl.when(above == 0)
    def _fallback():
        iota = lax.broadcasted_iota(jnp.int32, (N, 1), 0)
        rem = conf
        fb = jnp.zeros((N, 1), jnp.bool_)
        for _ in range(5):
            m = jnp.max(rem)
            j = jnp.min(jnp.where(rem == m, iota, jnp.int32(N)))
            hit = iota == j
            fb = fb | hit
            rem = jnp.where(hit, NEG_INF, rem)
        final_fb = fb & valid
        mask_ref[0] = final_fb
        scores_ref[0] = jnp.where(final_fb, conf, NEG_INF)


@jax.jit
def kernel(prediction):
    B, N, D = prediction.shape
    det, mask, scores = pl.pallas_call(
        _body,
        grid=(B,),
        in_specs=[pl.BlockSpec((1, N, D), lambda b: (b, 0, 0))],
        out_specs=[
            pl.BlockSpec((1, N, 7), lambda b: (b, 0, 0)),
            pl.BlockSpec((1, N, 1), lambda b: (b, 0, 0)),
            pl.BlockSpec((1, N, 1), lambda b: (b, 0, 0)),
        ],
        out_shape=[
            jax.ShapeDtypeStruct((B, N, 7), jnp.float32),
            jax.ShapeDtypeStruct((B, N, 1), jnp.bool_),
            jax.ShapeDtypeStruct((B, N, 1), jnp.float32),
        ],
    )(prediction)
    return det, mask[:, :, 0], scores[:, :, 0]
